# baseline (speedup 1.0000x reference)
import dataclasses

import jax
import jax.numpy as jnp
from jax import lax
from jax.experimental import pallas as pl
from jax.experimental.pallas import tpu as pltpu
from jax.experimental.pallas import tpu_sc as plsc

NB_TF = 4
B, C, H, W = 256, 3, 224, 224
L = 16
NC, NS = 2, 16
NW = NC * NS
KCH = B // L
HPT = H // NW
HALF = H // 2


def _minmax_body(xt_hbm, part_hbm, buf0, buf1, acc_v, stage_v, shared_v,
                 isem0, isem1):
    cid = lax.axis_index("c")
    sid = lax.axis_index("s")
    wid = cid * NS + sid

    bufs = (buf0, buf1)
    isems = (isem0, isem1)

    def load(j, b):
        c, r = divmod(j, HPT)
        pltpu.async_copy(xt_hbm.at[c, wid * HPT + r], bufs[b], isems[b])

    def wait_load(b):
        pltpu.make_async_copy(xt_hbm.at[0, 0], bufs[b], isems[b]).wait()

    for c in range(C):
        @pl.loop(0, KCH)
        def _(k):
            acc_v[c, 0, pl.ds(k * L, L)] = jnp.full((L,), jnp.inf, jnp.float32)
            acc_v[c, 1, pl.ds(k * L, L)] = jnp.full((L,), -jnp.inf, jnp.float32)

    load(0, 0)
    for j in range(C * HPT):
        b = j % 2
        c = j // HPT
        wait_load(b)
        if j + 1 < C * HPT:
            load(j + 1, 1 - b)

        @pl.loop(0, KCH)
        def _(k):
            sl = pl.ds(k * L, L)

            def w_body(w, carry):
                mn, mx = carry
                v = bufs[b][w, sl]
                return jnp.minimum(mn, v), jnp.maximum(mx, v)

            mn, mx = lax.fori_loop(
                0, H, w_body,
                (jnp.full((L,), jnp.inf, jnp.float32),
                 jnp.full((L,), -jnp.inf, jnp.float32)),
            )
            acc_v[c, 0, sl] = jnp.minimum(acc_v[c, 0, sl], mn)
            acc_v[c, 1, sl] = jnp.maximum(acc_v[c, 1, sl], mx)

    pltpu.sync_copy(acc_v, shared_v.at[sid])
    plsc.subcore_barrier()

    @pl.when(sid == 0)
    def _():
        @pl.loop(0, NS)
        def _(i):
            pltpu.sync_copy(shared_v.at[i], stage_v)
            for c in range(C):
                @pl.loop(0, KCH)
                def _(k):
                    sl = pl.ds(k * L, L)
                    acc_v[c, 0, sl] = jnp.minimum(acc_v[c, 0, sl], stage_v[c, 0, sl])
                    acc_v[c, 1, sl] = jnp.maximum(acc_v[c, 1, sl], stage_v[c, 1, sl])

        pltpu.sync_copy(acc_v, part_hbm.at[cid])


def _apply_body(xt_hbm, tf_hbm, part_hbm, o_hbm, bufA, bufB, cons_v,
                stage2_v, tf_v, semA, semB, osemA, osemB):
    cid = lax.axis_index("c")
    sid = lax.axis_index("s")
    wid = cid * NS + sid

    pltpu.sync_copy(tf_hbm, tf_v)
    pltpu.sync_copy(part_hbm, stage2_v)
    for c in range(C):
        @pl.loop(0, KCH)
        def _(k):
            sl = pl.ds(k * L, L)
            mn = jnp.minimum(stage2_v[0, c, 0, sl], stage2_v[1, c, 0, sl])
            mx = jnp.maximum(stage2_v[0, c, 1, sl], stage2_v[1, c, 1, sl])
            sc = jnp.full((L,), 1.0, jnp.float32) / jnp.maximum(
                mx - mn, jnp.full((L,), 1e-6, jnp.float32))
            cons_v[c, 0, sl] = mn
            cons_v[c, 1, sl] = sc

    for c in range(C):
        @pl.loop(0, HALF // NW + 1)
        def _(k4):
            hp = wid + NW * k4

            @pl.when(hp < HALF)
            def _():
                mhp = H - 1 - hp
                pltpu.async_copy(xt_hbm.at[c, hp], bufA, semA)
                pltpu.async_copy(xt_hbm.at[c, mhp], bufB, semB)
                pltpu.make_async_copy(xt_hbm.at[0, 0], bufA, semA).wait()
                pltpu.make_async_copy(xt_hbm.at[0, 0], bufB, semB).wait()

                @pl.loop(0, KCH)
                def _(k):
                    sl = pl.ds(k * L, L)
                    tv = tf_v[sl]
                    m1 = tv == 1
                    m2 = tv == 2
                    m3 = tv == 3
                    mn = cons_v[c, 0, sl]
                    sc = cons_v[c, 1, sl]

                    @pl.loop(0, HALF)
                    def _(w):
                        mw = W - 1 - w
                        aw = bufA[w, sl]
                        am = bufA[mw, sl]
                        bw = bufB[w, sl]
                        bm = bufB[mw, sl]
                        bufA[w, sl] = jnp.where(
                            m3, (aw - mn) * sc,
                            jnp.where(m2, bw, jnp.where(m1, am, aw)))
                        bufA[mw, sl] = jnp.where(
                            m3, (am - mn) * sc,
                            jnp.where(m2, bm, jnp.where(m1, aw, am)))
                        bufB[w, sl] = jnp.where(
                            m3, (bw - mn) * sc,
                            jnp.where(m2, aw, jnp.where(m1, bm, bw)))
                        bufB[mw, sl] = jnp.where(
                            m3, (bm - mn) * sc,
                            jnp.where(m2, am, jnp.where(m1, bw, bm)))

                pltpu.async_copy(bufA, o_hbm.at[c, hp], osemA)
                pltpu.async_copy(bufB, o_hbm.at[c, mhp], osemB)
                pltpu.make_async_copy(bufA, o_hbm.at[0, 0], osemA).wait()
                pltpu.make_async_copy(bufB, o_hbm.at[0, 0], osemB).wait()


@jax.jit
def kernel(x, sampled_tf):
    xt = jnp.transpose(x, (1, 2, 3, 0))
    mesh = plsc.VectorSubcoreMesh(
        core_axis_name="c", subcore_axis_name="s", num_cores=NC, num_subcores=NS
    )
    cp = pltpu.CompilerParams()
    if "needs_layout_passes" in pltpu.CompilerParams.__dataclass_fields__:
        cp = dataclasses.replace(cp, needs_layout_passes=False)
    part = pl.kernel(
        _minmax_body,
        out_type=jax.ShapeDtypeStruct((NC, C, 2, B), jnp.float32),
        mesh=mesh,
        scratch_types=[
            pltpu.VMEM((W, B), jnp.float32),
            pltpu.VMEM((W, B), jnp.float32),
            pltpu.VMEM((C, 2, B), jnp.float32),
            pltpu.VMEM((C, 2, B), jnp.float32),
            pltpu.VMEM_SHARED((NS, C, 2, B), jnp.float32),
            pltpu.SemaphoreType.DMA,
            pltpu.SemaphoreType.DMA,
        ],
        compiler_params=cp,
    )(xt)
    ot = pl.kernel(
        _apply_body,
        out_type=jax.ShapeDtypeStruct((C, H, W, B), jnp.float32),
        mesh=mesh,
        scratch_types=[
            pltpu.VMEM((W, B), jnp.float32),
            pltpu.VMEM((W, B), jnp.float32),
            pltpu.VMEM((C, 2, B), jnp.float32),
            pltpu.VMEM((NC, C, 2, B), jnp.float32),
            pltpu.VMEM((B,), jnp.int32),
            pltpu.SemaphoreType.DMA,
            pltpu.SemaphoreType.DMA,
            pltpu.SemaphoreType.DMA,
            pltpu.SemaphoreType.DMA,
        ],
        compiler_params=cp,
    )(xt, sampled_tf, part)
    return jnp.transpose(ot, (3, 0, 1, 2))


# speedup vs baseline: 1.6088x; 1.6088x over previous
import dataclasses

import jax
import jax.numpy as jnp
from jax import lax
from jax.experimental import pallas as pl
from jax.experimental.pallas import tpu as pltpu
from jax.experimental.pallas import tpu_sc as plsc

NB_TF = 4
B, C, H, W = 256, 3, 224, 224
L = 16
NC, NS = 2, 16
NW = NC * NS
KCH = B // L
HPT = H // NW
HALF = H // 2


def _minmax_body(xt_hbm, part_hbm, buf0, buf1, acc_v, stage_v, shared_v,
                 isem0, isem1):
    cid = lax.axis_index("c")
    sid = lax.axis_index("s")
    wid = cid * NS + sid

    bufs = (buf0, buf1)
    isems = (isem0, isem1)

    def load(j, b):
        c, r = divmod(j, HPT)
        pltpu.async_copy(xt_hbm.at[c, wid * HPT + r], bufs[b], isems[b])

    def wait_load(b):
        pltpu.make_async_copy(xt_hbm.at[0, 0], bufs[b], isems[b]).wait()

    for c in range(C):
        @pl.loop(0, KCH)
        def _(k):
            acc_v[c, 0, pl.ds(k * L, L)] = jnp.full((L,), jnp.inf, jnp.float32)
            acc_v[c, 1, pl.ds(k * L, L)] = jnp.full((L,), -jnp.inf, jnp.float32)

    load(0, 0)
    for j in range(C * HPT):
        b = j % 2
        c = j // HPT
        wait_load(b)
        if j + 1 < C * HPT:
            load(j + 1, 1 - b)

        @pl.loop(0, KCH)
        def _(k):
            sl = pl.ds(k * L, L)

            @pl.loop(0, H, step=16)
            def _(w0):
                vs = [bufs[b][w0 + i, sl] for i in range(16)]
                mn = vs[0]
                mx = vs[0]
                for v in vs[1:]:
                    mn = jnp.minimum(mn, v)
                    mx = jnp.maximum(mx, v)
                acc_v[c, 0, sl] = jnp.minimum(acc_v[c, 0, sl], mn)
                acc_v[c, 1, sl] = jnp.maximum(acc_v[c, 1, sl], mx)

    pltpu.sync_copy(acc_v, shared_v.at[sid])
    plsc.subcore_barrier()

    @pl.when(sid == 0)
    def _():
        @pl.loop(0, NS)
        def _(i):
            pltpu.sync_copy(shared_v.at[i], stage_v)
            for c in range(C):
                @pl.loop(0, KCH)
                def _(k):
                    sl = pl.ds(k * L, L)
                    acc_v[c, 0, sl] = jnp.minimum(acc_v[c, 0, sl], stage_v[c, 0, sl])
                    acc_v[c, 1, sl] = jnp.maximum(acc_v[c, 1, sl], stage_v[c, 1, sl])

        pltpu.sync_copy(acc_v, part_hbm.at[cid])


def _apply_body(xt_hbm, tf_hbm, part_hbm, o_hbm, bufA, bufB, cons_v,
                stage2_v, tf_v, semA, semB, osemA, osemB):
    cid = lax.axis_index("c")
    sid = lax.axis_index("s")
    wid = cid * NS + sid

    pltpu.sync_copy(tf_hbm, tf_v)
    pltpu.sync_copy(part_hbm, stage2_v)
    for c in range(C):
        @pl.loop(0, KCH)
        def _(k):
            sl = pl.ds(k * L, L)
            mn = jnp.minimum(stage2_v[0, c, 0, sl], stage2_v[1, c, 0, sl])
            mx = jnp.maximum(stage2_v[0, c, 1, sl], stage2_v[1, c, 1, sl])
            sc = jnp.full((L,), 1.0, jnp.float32) / jnp.maximum(
                mx - mn, jnp.full((L,), 1e-6, jnp.float32))
            cons_v[c, 0, sl] = mn
            cons_v[c, 1, sl] = sc

    for c in range(C):
        wsel = wid if c % 2 == 0 else NW - 1 - wid

        @pl.loop(0, HALF // NW + 1)
        def _(k4):
            hp = wsel + NW * k4

            @pl.when(hp < HALF)
            def _():
                mhp = H - 1 - hp
                pltpu.async_copy(xt_hbm.at[c, hp], bufA, semA)
                pltpu.async_copy(xt_hbm.at[c, mhp], bufB, semB)
                pltpu.make_async_copy(xt_hbm.at[0, 0], bufA, semA).wait()
                pltpu.make_async_copy(xt_hbm.at[0, 0], bufB, semB).wait()

                @pl.loop(0, KCH)
                def _(k):
                    sl = pl.ds(k * L, L)
                    tv = tf_v[sl]
                    m1 = tv == 1
                    m2 = tv == 2
                    m3 = tv == 3
                    mn = cons_v[c, 0, sl]
                    sc = cons_v[c, 1, sl]

                    @pl.loop(0, HALF, step=4)
                    def _(w0):
                        for i in range(4):
                            w = w0 + i
                            mw = W - 1 - w
                            aw = bufA[w, sl]
                            am = bufA[mw, sl]
                            bw = bufB[w, sl]
                            bm = bufB[mw, sl]
                            bufA[w, sl] = jnp.where(
                                m3, (aw - mn) * sc,
                                jnp.where(m2, bw, jnp.where(m1, am, aw)))
                            bufA[mw, sl] = jnp.where(
                                m3, (am - mn) * sc,
                                jnp.where(m2, bm, jnp.where(m1, aw, am)))
                            bufB[w, sl] = jnp.where(
                                m3, (bw - mn) * sc,
                                jnp.where(m2, aw, jnp.where(m1, bm, bw)))
                            bufB[mw, sl] = jnp.where(
                                m3, (bm - mn) * sc,
                                jnp.where(m2, am, jnp.where(m1, bw, bm)))

                pltpu.async_copy(bufA, o_hbm.at[c, hp], osemA)
                pltpu.async_copy(bufB, o_hbm.at[c, mhp], osemB)
                pltpu.make_async_copy(bufA, o_hbm.at[0, 0], osemA).wait()
                pltpu.make_async_copy(bufB, o_hbm.at[0, 0], osemB).wait()


@jax.jit
def kernel(x, sampled_tf):
    xt = jnp.transpose(x, (1, 2, 3, 0))
    mesh = plsc.VectorSubcoreMesh(
        core_axis_name="c", subcore_axis_name="s", num_cores=NC, num_subcores=NS
    )
    cp = pltpu.CompilerParams()
    if "needs_layout_passes" in pltpu.CompilerParams.__dataclass_fields__:
        cp = dataclasses.replace(cp, needs_layout_passes=False)
    part = pl.kernel(
        _minmax_body,
        out_type=jax.ShapeDtypeStruct((NC, C, 2, B), jnp.float32),
        mesh=mesh,
        scratch_types=[
            pltpu.VMEM((W, B), jnp.float32),
            pltpu.VMEM((W, B), jnp.float32),
            pltpu.VMEM((C, 2, B), jnp.float32),
            pltpu.VMEM((C, 2, B), jnp.float32),
            pltpu.VMEM_SHARED((NS, C, 2, B), jnp.float32),
            pltpu.SemaphoreType.DMA,
            pltpu.SemaphoreType.DMA,
        ],
        compiler_params=cp,
    )(xt)
    ot = pl.kernel(
        _apply_body,
        out_type=jax.ShapeDtypeStruct((C, H, W, B), jnp.float32),
        mesh=mesh,
        scratch_types=[
            pltpu.VMEM((W, B), jnp.float32),
            pltpu.VMEM((W, B), jnp.float32),
            pltpu.VMEM((C, 2, B), jnp.float32),
            pltpu.VMEM((NC, C, 2, B), jnp.float32),
            pltpu.VMEM((B,), jnp.int32),
            pltpu.SemaphoreType.DMA,
            pltpu.SemaphoreType.DMA,
            pltpu.SemaphoreType.DMA,
            pltpu.SemaphoreType.DMA,
        ],
        compiler_params=cp,
    )(xt, sampled_tf, part)
    return jnp.transpose(ot, (3, 0, 1, 2))


# speedup vs baseline: 1.8565x; 1.1540x over previous
import dataclasses

import jax
import jax.numpy as jnp
from jax import lax
from jax.experimental import pallas as pl
from jax.experimental.pallas import tpu as pltpu
from jax.experimental.pallas import tpu_sc as plsc

NB_TF = 4
B, C, H, W = 256, 3, 224, 224
L = 16
NC, NS = 2, 16
NW = NC * NS
KCH = B // L
HPT = H // NW
HALF = H // 2


def _minmax_body(xt_hbm, part_hbm, buf0, buf1, acc_v, stage_v, shared_v,
                 isem0, isem1):
    cid = lax.axis_index("c")
    sid = lax.axis_index("s")
    wid = cid * NS + sid

    bufs = (buf0, buf1)
    isems = (isem0, isem1)

    def load(j, b):
        c, r = divmod(j, HPT)
        pltpu.async_copy(xt_hbm.at[c, wid * HPT + r], bufs[b], isems[b])

    def wait_load(b):
        pltpu.make_async_copy(xt_hbm.at[0, 0], bufs[b], isems[b]).wait()

    for c in range(C):
        @pl.loop(0, KCH)
        def _(k):
            acc_v[c, 0, pl.ds(k * L, L)] = jnp.full((L,), jnp.inf, jnp.float32)
            acc_v[c, 1, pl.ds(k * L, L)] = jnp.full((L,), -jnp.inf, jnp.float32)

    load(0, 0)
    for j in range(C * HPT):
        b = j % 2
        c = j // HPT
        wait_load(b)
        if j + 1 < C * HPT:
            load(j + 1, 1 - b)

        @pl.loop(0, KCH)
        def _(k):
            sl = pl.ds(k * L, L)

            @pl.loop(0, H, step=16)
            def _(w0):
                vs = [bufs[b][w0 + i, sl] for i in range(16)]
                mn = vs[0]
                mx = vs[0]
                for v in vs[1:]:
                    mn = jnp.minimum(mn, v)
                    mx = jnp.maximum(mx, v)
                acc_v[c, 0, sl] = jnp.minimum(acc_v[c, 0, sl], mn)
                acc_v[c, 1, sl] = jnp.maximum(acc_v[c, 1, sl], mx)

    pltpu.sync_copy(acc_v, shared_v.at[sid])
    plsc.subcore_barrier()

    @pl.when(sid == 0)
    def _():
        @pl.loop(0, NS)
        def _(i):
            pltpu.sync_copy(shared_v.at[i], stage_v)
            for c in range(C):
                @pl.loop(0, KCH)
                def _(k):
                    sl = pl.ds(k * L, L)
                    acc_v[c, 0, sl] = jnp.minimum(acc_v[c, 0, sl], stage_v[c, 0, sl])
                    acc_v[c, 1, sl] = jnp.maximum(acc_v[c, 1, sl], stage_v[c, 1, sl])

        pltpu.sync_copy(acc_v, part_hbm.at[cid])


LH = B // 2
ITEMS_PER_C = 2 * HALF // NW


def _apply_body(xt_hbm, tf_hbm, part_hbm, o_hbm,
                bufA0, bufB0, bufA1, bufB1, cons_v, stage2_v, tf_v,
                iA0, iB0, iA1, iB1, oA0, oB0, oA1, oB1):
    cid = lax.axis_index("c")
    sid = lax.axis_index("s")
    wid = cid * NS + sid

    bufsA = (bufA0, bufA1)
    bufsB = (bufB0, bufB1)
    isemsA = (iA0, iA1)
    isemsB = (iB0, iB1)
    osemsA = (oA0, oA1)
    osemsB = (oB0, oB1)

    pltpu.sync_copy(tf_hbm, tf_v)
    pltpu.sync_copy(part_hbm, stage2_v)
    for c in range(C):
        @pl.loop(0, KCH)
        def _(k):
            sl = pl.ds(k * L, L)
            mn = jnp.minimum(stage2_v[0, c, 0, sl], stage2_v[1, c, 0, sl])
            mx = jnp.maximum(stage2_v[0, c, 1, sl], stage2_v[1, c, 1, sl])
            sc = jnp.full((L,), 1.0, jnp.float32) / jnp.maximum(
                mx - mn, jnp.full((L,), 1e-6, jnp.float32))
            cons_v[c, 0, sl] = mn
            cons_v[c, 1, sl] = sc

    NITEMS = C * ITEMS_PER_C

    def item_coords(m):
        c = m // ITEMS_PER_C
        q = wid + NW * (m % ITEMS_PER_C)
        hp = lax.shift_right_logical(q, 1)
        g = lax.bitwise_and(q, 1)
        return c, hp, g

    def load(m, b):
        c, hp, g = item_coords(m)
        col = pl.ds(g * LH, LH)
        pltpu.async_copy(xt_hbm.at[c, hp, :, col], bufsA[b], isemsA[b])
        pltpu.async_copy(xt_hbm.at[c, H - 1 - hp, :, col], bufsB[b], isemsB[b])

    def store(m, b):
        c, hp, g = item_coords(m)
        col = pl.ds(g * LH, LH)
        pltpu.async_copy(bufsA[b], o_hbm.at[c, hp, :, col], osemsA[b])
        pltpu.async_copy(bufsB[b], o_hbm.at[c, H - 1 - hp, :, col], osemsB[b])

    def wait_load(b):
        pltpu.make_async_copy(xt_hbm.at[0, 0, :, pl.ds(0, LH)], bufsA[b], isemsA[b]).wait()
        pltpu.make_async_copy(xt_hbm.at[0, 0, :, pl.ds(0, LH)], bufsB[b], isemsB[b]).wait()

    def wait_store(b):
        pltpu.make_async_copy(bufsA[b], o_hbm.at[0, 0, :, pl.ds(0, LH)], osemsA[b]).wait()
        pltpu.make_async_copy(bufsB[b], o_hbm.at[0, 0, :, pl.ds(0, LH)], osemsB[b]).wait()

    load(0, 0)
    for m in range(NITEMS):
        b = m % 2
        c, hp, g = item_coords(m)
        gbase = g * LH
        wait_load(b)

        @pl.loop(0, LH // L)
        def _(kk):
            sl = pl.ds(kk * L, L)
            gsl = pl.ds(gbase + kk * L, L)
            tv = tf_v[gsl]
            m1 = tv == 1
            m2 = tv == 2
            m3 = tv == 3
            mn = cons_v[c, 0, gsl]
            sc = cons_v[c, 1, gsl]
            bA = bufsA[b]
            bB = bufsB[b]

            @pl.loop(0, HALF, step=4)
            def _(w0):
                for i in range(4):
                    w = w0 + i
                    mw = W - 1 - w
                    aw = bA[w, sl]
                    am = bA[mw, sl]
                    bw = bB[w, sl]
                    bm = bB[mw, sl]
                    bA[w, sl] = jnp.where(
                        m3, (aw - mn) * sc,
                        jnp.where(m2, bw, jnp.where(m1, am, aw)))
                    bA[mw, sl] = jnp.where(
                        m3, (am - mn) * sc,
                        jnp.where(m2, bm, jnp.where(m1, aw, am)))
                    bB[w, sl] = jnp.where(
                        m3, (bw - mn) * sc,
                        jnp.where(m2, aw, jnp.where(m1, bm, bw)))
                    bB[mw, sl] = jnp.where(
                        m3, (bm - mn) * sc,
                        jnp.where(m2, am, jnp.where(m1, bw, bm)))

        if m >= 1:
            wait_store(1 - b)
        if m + 1 < NITEMS:
            load(m + 1, 1 - b)
        store(m, b)

    wait_store((NITEMS - 1) % 2)


@jax.jit
def kernel(x, sampled_tf):
    xt = jnp.transpose(x, (1, 2, 3, 0))
    mesh = plsc.VectorSubcoreMesh(
        core_axis_name="c", subcore_axis_name="s", num_cores=NC, num_subcores=NS
    )
    cp = pltpu.CompilerParams()
    if "needs_layout_passes" in pltpu.CompilerParams.__dataclass_fields__:
        cp = dataclasses.replace(cp, needs_layout_passes=False)
    part = pl.kernel(
        _minmax_body,
        out_type=jax.ShapeDtypeStruct((NC, C, 2, B), jnp.float32),
        mesh=mesh,
        scratch_types=[
            pltpu.VMEM((W, B), jnp.float32),
            pltpu.VMEM((W, B), jnp.float32),
            pltpu.VMEM((C, 2, B), jnp.float32),
            pltpu.VMEM((C, 2, B), jnp.float32),
            pltpu.VMEM_SHARED((NS, C, 2, B), jnp.float32),
            pltpu.SemaphoreType.DMA,
            pltpu.SemaphoreType.DMA,
        ],
        compiler_params=cp,
    )(xt)
    ot = pl.kernel(
        _apply_body,
        out_type=jax.ShapeDtypeStruct((C, H, W, B), jnp.float32),
        mesh=mesh,
        scratch_types=[
            pltpu.VMEM((W, LH), jnp.float32),
            pltpu.VMEM((W, LH), jnp.float32),
            pltpu.VMEM((W, LH), jnp.float32),
            pltpu.VMEM((W, LH), jnp.float32),
            pltpu.VMEM((C, 2, B), jnp.float32),
            pltpu.VMEM((NC, C, 2, B), jnp.float32),
            pltpu.VMEM((B,), jnp.int32),
            pltpu.SemaphoreType.DMA,
            pltpu.SemaphoreType.DMA,
            pltpu.SemaphoreType.DMA,
            pltpu.SemaphoreType.DMA,
            pltpu.SemaphoreType.DMA,
            pltpu.SemaphoreType.DMA,
            pltpu.SemaphoreType.DMA,
            pltpu.SemaphoreType.DMA,
        ],
        compiler_params=cp,
    )(xt, sampled_tf, part)
    return jnp.transpose(ot, (3, 0, 1, 2))
